# baseline (speedup 1.0000x reference)
import functools

import jax
import jax.numpy as jnp
from jax import lax
from jax.experimental import pallas as pl
from jax.experimental.pallas import tpu as pltpu
from jax.experimental.pallas import tpu_sc as plsc

_NC = 2
_NS = 16
_L = 16
_CH = 128
_CW = 16
_SK0 = 3
_SKD = 8


def _sc_aggregate(n_pad, d, k0, k1):
  n_chunks = max(k0, k1)
  rows_pt = n_pad // _NS
  mesh = plsc.VectorSubcoreMesh(
      core_axis_name="c", subcore_axis_name="s",
      num_cores=_NC, num_subcores=_NS)

  out_type = [jax.ShapeDtypeStruct((_NC, n_pad, d), jnp.float32)]
  scratch = [
      pltpu.VMEM((n_chunks, _CH), jnp.int32),
      pltpu.VMEM((n_chunks, _CH), jnp.int32),
      pltpu.VMEM((_CH, d), jnp.float32),
      pltpu.VMEM_SHARED((n_pad, d), jnp.float32),
      pltpu.SemaphoreType.DMA,
  ]

  def body(x_hbm, src_hbm, dst_hbm, zeros_hbm, sums_hbm, sidx, didx, rows,
           acc, sem):
    c = lax.axis_index("c")
    s = lax.axis_index("s")
    wid = c * _NS + s

    r0 = s * rows_pt
    pltpu.sync_copy(zeros_hbm.at[pl.ds(r0, rows_pt)],
                    acc.at[pl.ds(r0, rows_pt)])
    plsc.subcore_barrier()

    pltpu.sync_copy(src_hbm.at[wid], sidx)
    pltpu.sync_copy(dst_hbm.at[wid], didx)

    def chunk(j, carry):
      pltpu.async_copy(x_hbm.at[sidx.at[j]], rows, sem).wait()
      pltpu.sync_copy(rows, acc.at[didx.at[j]], add=True)
      return carry

    nj = jnp.where(c == 0, k0, k1)
    lax.fori_loop(0, nj, chunk, 0)
    plsc.subcore_barrier()

    pltpu.sync_copy(acc.at[pl.ds(r0, rows_pt)],
                    sums_hbm.at[c, pl.ds(r0, rows_pt)])

  return pl.kernel(body, out_type=out_type, mesh=mesh, scratch_types=scratch)


def _sc_counts(n_pad, d, n_chunks):
  rows_pt = n_pad // _NS
  mesh = plsc.VectorSubcoreMesh(
      core_axis_name="c", subcore_axis_name="s",
      num_cores=_NC, num_subcores=_NS)

  out_type = [jax.ShapeDtypeStruct((_NC, n_pad, d), jnp.float32)]
  scratch = [
      pltpu.VMEM((n_chunks, _CH), jnp.int32),
      pltpu.VMEM((_CH, d), jnp.float32),
      pltpu.VMEM_SHARED((n_pad, d), jnp.float32),
  ]

  def body(dst_hbm, zeros_hbm, cnts_hbm, didx, ones, cacc):
    c = lax.axis_index("c")
    s = lax.axis_index("s")
    wid = c * _NS + s

    zv = jnp.zeros((_L,), jnp.float32)

    def fillones(i, carry):
      for j in range(d // _L):
        ones[i, pl.ds(j * _L, _L)] = zv + 1.0
      return carry

    lax.fori_loop(0, _CH, fillones, 0)

    r0 = s * rows_pt
    pltpu.sync_copy(zeros_hbm.at[pl.ds(r0, rows_pt)],
                    cacc.at[pl.ds(r0, rows_pt)])
    plsc.subcore_barrier()

    pltpu.sync_copy(dst_hbm.at[wid], didx)

    def chunk(j, carry):
      pltpu.sync_copy(ones, cacc.at[didx.at[j]], add=True)
      return carry

    lax.fori_loop(0, n_chunks, chunk, 0)
    plsc.subcore_barrier()

    pltpu.sync_copy(cacc.at[pl.ds(r0, rows_pt)],
                    cnts_hbm.at[c, pl.ds(r0, rows_pt)])

  return pl.kernel(body, out_type=out_type, mesh=mesh, scratch_types=scratch)


def _dense_layer_body(s0_ref, s1_ref, c0_ref, c1_ref, x_ref, wl_ref, wr_ref,
                      b_ref, h_ref):
  cnt = c0_ref[...][:, :1] + c1_ref[...][:, :1]
  rinv = 1.0 / jnp.maximum(cnt, 1.0)
  mean = (s0_ref[...] + s1_ref[...]) * rinv
  h_ref[...] = (
      jnp.dot(mean, wl_ref[...], preferred_element_type=jnp.float32)
      + jnp.dot(x_ref[...], wr_ref[...], preferred_element_type=jnp.float32)
      + b_ref[...])


def _dense_head_body(s0_ref, s1_ref, c0_ref, c1_ref, x_ref, wl_ref, wr_ref,
                     b_ref, wout_ref, out_ref):
  cnt = c0_ref[...][:, :1] + c1_ref[...][:, :1]
  rinv = 1.0 / jnp.maximum(cnt, 1.0)
  mean = (s0_ref[...] + s1_ref[...]) * rinv
  h = (jnp.dot(mean, wl_ref[...], preferred_element_type=jnp.float32)
       + jnp.dot(x_ref[...], wr_ref[...], preferred_element_type=jnp.float32)
       + b_ref[...])
  logits = jnp.dot(h, wout_ref[...], preferred_element_type=jnp.float32)
  m = jnp.max(logits, axis=1, keepdims=True)
  z = logits - m
  lse = jnp.log(jnp.sum(jnp.exp(z), axis=1, keepdims=True))
  out_ref[...] = z - lse


def _dense_call(body, n_pad, bn, d, out_dim, extra_w):
  grid = (n_pad // bn,)
  row_spec = pl.BlockSpec((bn, d), lambda i: (i, 0))
  cnt_spec = pl.BlockSpec((bn, d), lambda i: (i, 0))
  w_spec = pl.BlockSpec((d, d), lambda i: (0, 0))
  b_spec = pl.BlockSpec((1, d), lambda i: (0, 0))
  in_specs = [row_spec, row_spec, cnt_spec, cnt_spec, row_spec,
              w_spec, w_spec, b_spec]
  if extra_w:
    in_specs.append(pl.BlockSpec((d, out_dim), lambda i: (0, 0)))
  return pl.pallas_call(
      body,
      grid=grid,
      in_specs=in_specs,
      out_specs=pl.BlockSpec((bn, out_dim), lambda i: (i, 0)),
      out_shape=jax.ShapeDtypeStruct((n_pad, out_dim), jnp.float32),
  )


def kernel(x, edge_index, nodes, Wl1, Wr1, b1, Wl2, Wr2, b2, Wout):
  n, d = x.shape
  e = edge_index.shape[1]
  out_dim = Wout.shape[1]
  nw = _NC * _NS

  n_pad = ((n + 1 + _NS * _L - 1) // (_NS * _L)) * (_NS * _L)
  cgrp = _SKD * _NS
  total_chunks = ((e + _CH - 1) // _CH + cgrp - 1) // cgrp * cgrp
  k0 = total_chunks * _SK0 // _SKD // _NS
  k1 = total_chunks // _NS - k0
  e_pad = total_chunks * _CH
  n_chunks = max(k0, k1)

  x_pad = jnp.concatenate(
      [x, jnp.zeros((n_pad - n, d), jnp.float32)], axis=0)
  pad_idx = jnp.full((e_pad - e,), n, jnp.int32)

  def skewed(idx):
    ch = jnp.concatenate([idx, pad_idx]).reshape(total_chunks, _CH)
    p0 = ch[:_NS * k0].reshape(_NS, k0, _CH)
    p0 = jnp.concatenate(
        [p0, jnp.full((_NS, n_chunks - k0, _CH), n, jnp.int32)], axis=1)
    p1 = ch[_NS * k0:].reshape(_NS, k1, _CH)
    p1 = jnp.concatenate(
        [p1, jnp.full((_NS, n_chunks - k1, _CH), n, jnp.int32)], axis=1)
    return jnp.concatenate([p0, p1], axis=0)

  src3 = skewed(edge_index[0])
  dst3 = skewed(edge_index[1])
  epw_u = ((e + nw - 1) // nw + _CH - 1) // _CH * _CH
  nc_u = epw_u // _CH
  pad_u = jnp.full((epw_u * nw - e,), n, jnp.int32)
  dst3_u = jnp.concatenate([edge_index[1], pad_u]).reshape(nw, nc_u, _CH)

  zeros = jnp.zeros((n_pad, d), jnp.float32)
  (sums1,) = _sc_aggregate(n_pad, d, k0, k1)(x_pad, src3, dst3, zeros)
  (counts,) = _sc_counts(n_pad, d, nc_u)(dst3_u, zeros)

  bn = n_pad // 4
  dense1 = _dense_call(_dense_layer_body, n_pad, bn, d, d, False)
  h1 = dense1(sums1[0], sums1[1], counts[0], counts[1], x_pad,
              Wl1, Wr1, b1[None, :])

  (sums2,) = _sc_aggregate(n_pad, d, k0, k1)(h1, src3, dst3, zeros)

  head = _dense_call(_dense_head_body, n_pad, bn, d, out_dim, True)
  out = head(sums2[0], sums2[1], counts[0], counts[1], h1,
             Wl2, Wr2, b2[None, :], Wout)
  return out[:n]


# speedup vs baseline: 1.3427x; 1.3427x over previous
import functools

import jax
import jax.numpy as jnp
from jax import lax
from jax.experimental import pallas as pl
from jax.experimental.pallas import tpu as pltpu
from jax.experimental.pallas import tpu_sc as plsc

_NC = 2
_NS = 16
_L = 16
_CH = 128
_CW = 16
_SK0 = 5
_SKD = 8


def _sc_aggregate(n_pad, d, k0, k1):
  n_chunks = max(k0, k1)
  rows_pt = n_pad // _NS
  mesh = plsc.VectorSubcoreMesh(
      core_axis_name="c", subcore_axis_name="s",
      num_cores=_NC, num_subcores=_NS)

  out_type = [jax.ShapeDtypeStruct((_NC, n_pad, d), jnp.float32)]
  scratch = [
      pltpu.VMEM((n_chunks, _CH), jnp.int32),
      pltpu.VMEM((n_chunks, _CH), jnp.int32),
      pltpu.VMEM((_CH, d), jnp.float32),
      pltpu.VMEM_SHARED((n_pad, d), jnp.float32),
      pltpu.SemaphoreType.DMA,
  ]

  def body(x_hbm, src_hbm, dst_hbm, zeros_hbm, sums_hbm, sidx, didx, rows,
           acc, sem):
    c = lax.axis_index("c")
    s = lax.axis_index("s")
    wid = c * _NS + s

    r0 = s * rows_pt
    pltpu.sync_copy(zeros_hbm.at[pl.ds(r0, rows_pt)],
                    acc.at[pl.ds(r0, rows_pt)])
    plsc.subcore_barrier()

    pltpu.sync_copy(src_hbm.at[wid], sidx)
    pltpu.sync_copy(dst_hbm.at[wid], didx)

    def chunk(j, carry):
      pltpu.async_copy(x_hbm.at[sidx.at[j]], rows, sem).wait()
      pltpu.sync_copy(rows, acc.at[didx.at[j]], add=True)
      return carry

    nj = jnp.where(c == 0, k0, k1)
    lax.fori_loop(0, nj, chunk, 0)
    plsc.subcore_barrier()

    pltpu.sync_copy(acc.at[pl.ds(r0, rows_pt)],
                    sums_hbm.at[c, pl.ds(r0, rows_pt)])

  return pl.kernel(body, out_type=out_type, mesh=mesh, scratch_types=scratch)


def _sc_counts(n_pad, d, n_chunks):
  rows_pt = n_pad // _NS
  mesh = plsc.VectorSubcoreMesh(
      core_axis_name="c", subcore_axis_name="s",
      num_cores=_NC, num_subcores=_NS)

  out_type = [jax.ShapeDtypeStruct((_NC, n_pad, d), jnp.float32)]
  scratch = [
      pltpu.VMEM((n_chunks, _CH), jnp.int32),
      pltpu.VMEM((_CH, d), jnp.float32),
      pltpu.VMEM_SHARED((n_pad, d), jnp.float32),
  ]

  def body(dst_hbm, zeros_hbm, cnts_hbm, didx, ones, cacc):
    c = lax.axis_index("c")
    s = lax.axis_index("s")
    wid = c * _NS + s

    zv = jnp.zeros((_L,), jnp.float32)

    def fillones(i, carry):
      for j in range(d // _L):
        ones[i, pl.ds(j * _L, _L)] = zv + 1.0
      return carry

    lax.fori_loop(0, _CH, fillones, 0)

    r0 = s * rows_pt
    pltpu.sync_copy(zeros_hbm.at[pl.ds(r0, rows_pt)],
                    cacc.at[pl.ds(r0, rows_pt)])
    plsc.subcore_barrier()

    pltpu.sync_copy(dst_hbm.at[wid], didx)

    def chunk(j, carry):
      pltpu.sync_copy(ones, cacc.at[didx.at[j]], add=True)
      return carry

    lax.fori_loop(0, n_chunks, chunk, 0)
    plsc.subcore_barrier()

    pltpu.sync_copy(cacc.at[pl.ds(r0, rows_pt)],
                    cnts_hbm.at[c, pl.ds(r0, rows_pt)])

  return pl.kernel(body, out_type=out_type, mesh=mesh, scratch_types=scratch)


def _dense_layer_body(s0_ref, s1_ref, c0_ref, c1_ref, x_ref, wl_ref, wr_ref,
                      b_ref, h_ref):
  cnt = c0_ref[...][:, :1] + c1_ref[...][:, :1]
  rinv = 1.0 / jnp.maximum(cnt, 1.0)
  mean = (s0_ref[...] + s1_ref[...]) * rinv
  h_ref[...] = (
      jnp.dot(mean, wl_ref[...], preferred_element_type=jnp.float32)
      + jnp.dot(x_ref[...], wr_ref[...], preferred_element_type=jnp.float32)
      + b_ref[...])


def _dense_head_body(s0_ref, s1_ref, c0_ref, c1_ref, x_ref, wl_ref, wr_ref,
                     b_ref, wout_ref, out_ref):
  cnt = c0_ref[...][:, :1] + c1_ref[...][:, :1]
  rinv = 1.0 / jnp.maximum(cnt, 1.0)
  mean = (s0_ref[...] + s1_ref[...]) * rinv
  h = (jnp.dot(mean, wl_ref[...], preferred_element_type=jnp.float32)
       + jnp.dot(x_ref[...], wr_ref[...], preferred_element_type=jnp.float32)
       + b_ref[...])
  logits = jnp.dot(h, wout_ref[...], preferred_element_type=jnp.float32)
  m = jnp.max(logits, axis=1, keepdims=True)
  z = logits - m
  lse = jnp.log(jnp.sum(jnp.exp(z), axis=1, keepdims=True))
  out_ref[...] = z - lse


def _dense_call(body, n_pad, bn, d, out_dim, extra_w):
  grid = (n_pad // bn,)
  row_spec = pl.BlockSpec((bn, d), lambda i: (i, 0))
  cnt_spec = pl.BlockSpec((bn, d), lambda i: (i, 0))
  w_spec = pl.BlockSpec((d, d), lambda i: (0, 0))
  b_spec = pl.BlockSpec((1, d), lambda i: (0, 0))
  in_specs = [row_spec, row_spec, cnt_spec, cnt_spec, row_spec,
              w_spec, w_spec, b_spec]
  if extra_w:
    in_specs.append(pl.BlockSpec((d, out_dim), lambda i: (0, 0)))
  return pl.pallas_call(
      body,
      grid=grid,
      in_specs=in_specs,
      out_specs=pl.BlockSpec((bn, out_dim), lambda i: (i, 0)),
      out_shape=jax.ShapeDtypeStruct((n_pad, out_dim), jnp.float32),
  )


def kernel(x, edge_index, nodes, Wl1, Wr1, b1, Wl2, Wr2, b2, Wout):
  n, d = x.shape
  e = edge_index.shape[1]
  out_dim = Wout.shape[1]
  nw = _NC * _NS

  n_pad = ((n + 1 + _NS * _L - 1) // (_NS * _L)) * (_NS * _L)
  cgrp = _SKD * _NS
  total_chunks = ((e + _CH - 1) // _CH + cgrp - 1) // cgrp * cgrp
  k0 = total_chunks * _SK0 // _SKD // _NS
  k1 = total_chunks // _NS - k0
  e_pad = total_chunks * _CH
  n_chunks = max(k0, k1)

  x_pad = jnp.concatenate(
      [x, jnp.zeros((n_pad - n, d), jnp.float32)], axis=0)
  pad_idx = jnp.full((e_pad - e,), n, jnp.int32)

  def skewed(idx):
    ch = jnp.concatenate([idx, pad_idx]).reshape(total_chunks, _CH)
    p0 = ch[:_NS * k0].reshape(_NS, k0, _CH)
    p0 = jnp.concatenate(
        [p0, jnp.full((_NS, n_chunks - k0, _CH), n, jnp.int32)], axis=1)
    p1 = ch[_NS * k0:].reshape(_NS, k1, _CH)
    p1 = jnp.concatenate(
        [p1, jnp.full((_NS, n_chunks - k1, _CH), n, jnp.int32)], axis=1)
    return jnp.concatenate([p0, p1], axis=0)

  src3 = skewed(edge_index[0])
  dst3 = skewed(edge_index[1])
  epw_u = ((e + nw - 1) // nw + _CH - 1) // _CH * _CH
  nc_u = epw_u // _CH
  pad_u = jnp.full((epw_u * nw - e,), n, jnp.int32)
  dst3_u = jnp.concatenate([edge_index[1], pad_u]).reshape(nw, nc_u, _CH)

  zeros = jnp.zeros((n_pad, d), jnp.float32)
  (sums1,) = _sc_aggregate(n_pad, d, k0, k1)(x_pad, src3, dst3, zeros)
  (counts,) = _sc_counts(n_pad, d, nc_u)(dst3_u, zeros)

  bn = n_pad // 4
  dense1 = _dense_call(_dense_layer_body, n_pad, bn, d, d, False)
  h1 = dense1(sums1[0], sums1[1], counts[0], counts[1], x_pad,
              Wl1, Wr1, b1[None, :])

  (sums2,) = _sc_aggregate(n_pad, d, k0, k1)(h1, src3, dst3, zeros)

  head = _dense_call(_dense_head_body, n_pad, bn, d, out_dim, True)
  out = head(sums2[0], sums2[1], counts[0], counts[1], h1,
             Wl2, Wr2, b2[None, :], Wout)
  return out[:n]


# speedup vs baseline: 1.6571x; 1.2341x over previous
import functools

import jax
import jax.numpy as jnp
from jax import lax
from jax.experimental import pallas as pl
from jax.experimental.pallas import tpu as pltpu
from jax.experimental.pallas import tpu_sc as plsc

_NC = 2
_NS = 16
_L = 16
_CH = 128
_CW = 16
_ZR = 64


def _sc_aggregate(n_pad, d, n_chunks):
  rows_pt = n_pad // _NS
  mesh = plsc.VectorSubcoreMesh(
      core_axis_name="c", subcore_axis_name="s",
      num_cores=_NC, num_subcores=_NS)

  out_type = [jax.ShapeDtypeStruct((_NC, n_pad, d), jnp.float32)]
  scratch = [
      pltpu.VMEM((n_chunks, _CH), jnp.int32),
      pltpu.VMEM((n_chunks, _CH), jnp.int32),
      pltpu.VMEM((_CH, d), jnp.float32),
      pltpu.VMEM((_ZR, d), jnp.float32),
      pltpu.VMEM_SHARED((n_pad, d), jnp.float32),
      pltpu.SemaphoreType.DMA,
  ]

  def body(x_hbm, src_hbm, dst_hbm, sums_hbm, sidx, didx, rows, zbuf, acc,
           sem):
    c = lax.axis_index("c")
    s = lax.axis_index("s")
    wid = c * _NS + s

    zv = jnp.zeros((_L,), jnp.float32)

    def zrow(i, carry):
      for j in range(d // _L):
        zbuf[i, pl.ds(j * _L, _L)] = zv
      return carry

    lax.fori_loop(0, _ZR, zrow, 0)

    r0 = s * rows_pt
    off = 0
    while off < rows_pt:
      nr = min(_ZR, rows_pt - off)
      pltpu.sync_copy(zbuf.at[pl.ds(0, nr)], acc.at[pl.ds(r0 + off, nr)])
      off += nr
    plsc.subcore_barrier()

    pltpu.sync_copy(src_hbm.at[wid], sidx)
    pltpu.sync_copy(dst_hbm.at[wid], didx)

    def chunk(j, carry):
      pltpu.async_copy(x_hbm.at[sidx.at[j]], rows, sem).wait()
      pltpu.sync_copy(rows, acc.at[didx.at[j]], add=True)
      return carry

    lax.fori_loop(0, n_chunks, chunk, 0)
    plsc.subcore_barrier()

    pltpu.sync_copy(acc.at[pl.ds(r0, rows_pt)],
                    sums_hbm.at[c, pl.ds(r0, rows_pt)])

  return pl.kernel(body, out_type=out_type, mesh=mesh, scratch_types=scratch)


def _sc_counts(n_pad, d, n_chunks):
  rows_pt = n_pad // _NS
  mesh = plsc.VectorSubcoreMesh(
      core_axis_name="c", subcore_axis_name="s",
      num_cores=_NC, num_subcores=_NS)

  out_type = [jax.ShapeDtypeStruct((_NC, n_pad, d), jnp.float32)]
  scratch = [
      pltpu.VMEM((n_chunks, _CH), jnp.int32),
      pltpu.VMEM((_CH, d), jnp.float32),
      pltpu.VMEM((_ZR, d), jnp.float32),
      pltpu.VMEM_SHARED((n_pad, d), jnp.float32),
  ]

  def body(dst_hbm, cnts_hbm, didx, ones, zbuf, cacc):
    c = lax.axis_index("c")
    s = lax.axis_index("s")
    wid = c * _NS + s

    zv = jnp.zeros((_L,), jnp.float32)

    def fillones(i, carry):
      for j in range(d // _L):
        ones[i, pl.ds(j * _L, _L)] = zv + 1.0
      return carry

    def fillzero(i, carry):
      for j in range(d // _L):
        zbuf[i, pl.ds(j * _L, _L)] = zv
      return carry

    lax.fori_loop(0, _CH, fillones, 0)
    lax.fori_loop(0, _ZR, fillzero, 0)

    r0 = s * rows_pt
    off = 0
    while off < rows_pt:
      nr = min(_ZR, rows_pt - off)
      pltpu.sync_copy(zbuf.at[pl.ds(0, nr)], cacc.at[pl.ds(r0 + off, nr)])
      off += nr
    plsc.subcore_barrier()

    pltpu.sync_copy(dst_hbm.at[wid], didx)

    def chunk(j, carry):
      pltpu.sync_copy(ones, cacc.at[didx.at[j]], add=True)
      return carry

    lax.fori_loop(0, n_chunks, chunk, 0)
    plsc.subcore_barrier()

    pltpu.sync_copy(cacc.at[pl.ds(r0, rows_pt)],
                    cnts_hbm.at[c, pl.ds(r0, rows_pt)])

  return pl.kernel(body, out_type=out_type, mesh=mesh, scratch_types=scratch)


def _dense_layer_body(s0_ref, s1_ref, c0_ref, c1_ref, x_ref, wl_ref, wr_ref,
                      b_ref, h_ref):
  cnt = c0_ref[...][:, :1] + c1_ref[...][:, :1]
  rinv = 1.0 / jnp.maximum(cnt, 1.0)
  mean = (s0_ref[...] + s1_ref[...]) * rinv
  h_ref[...] = (
      jnp.dot(mean, wl_ref[...], preferred_element_type=jnp.float32)
      + jnp.dot(x_ref[...], wr_ref[...], preferred_element_type=jnp.float32)
      + b_ref[...])


def _dense_head_body(s0_ref, s1_ref, c0_ref, c1_ref, x_ref, wl_ref, wr_ref,
                     b_ref, wout_ref, out_ref):
  cnt = c0_ref[...][:, :1] + c1_ref[...][:, :1]
  rinv = 1.0 / jnp.maximum(cnt, 1.0)
  mean = (s0_ref[...] + s1_ref[...]) * rinv
  h = (jnp.dot(mean, wl_ref[...], preferred_element_type=jnp.float32)
       + jnp.dot(x_ref[...], wr_ref[...], preferred_element_type=jnp.float32)
       + b_ref[...])
  logits = jnp.dot(h, wout_ref[...], preferred_element_type=jnp.float32)
  m = jnp.max(logits, axis=1, keepdims=True)
  z = logits - m
  lse = jnp.log(jnp.sum(jnp.exp(z), axis=1, keepdims=True))
  out_ref[...] = z - lse


def _dense_call(body, n_pad, bn, d, out_dim, extra_w):
  grid = (n_pad // bn,)
  row_spec = pl.BlockSpec((bn, d), lambda i: (i, 0))
  cnt_spec = pl.BlockSpec((bn, d), lambda i: (i, 0))
  w_spec = pl.BlockSpec((d, d), lambda i: (0, 0))
  b_spec = pl.BlockSpec((1, d), lambda i: (0, 0))
  in_specs = [row_spec, row_spec, cnt_spec, cnt_spec, row_spec,
              w_spec, w_spec, b_spec]
  if extra_w:
    in_specs.append(pl.BlockSpec((d, out_dim), lambda i: (0, 0)))
  return pl.pallas_call(
      body,
      grid=grid,
      in_specs=in_specs,
      out_specs=pl.BlockSpec((bn, out_dim), lambda i: (i, 0)),
      out_shape=jax.ShapeDtypeStruct((n_pad, out_dim), jnp.float32),
  )


def kernel(x, edge_index, nodes, Wl1, Wr1, b1, Wl2, Wr2, b2, Wout):
  n, d = x.shape
  e = edge_index.shape[1]
  out_dim = Wout.shape[1]
  nw = _NC * _NS

  n_pad = ((n + 1 + _NS * _L - 1) // (_NS * _L)) * (_NS * _L)
  epw = ((e + nw - 1) // nw + _CH - 1) // _CH * _CH
  e_pad = epw * nw
  n_chunks = epw // _CH

  x_pad = jnp.concatenate(
      [x, jnp.zeros((n_pad - n, d), jnp.float32)], axis=0)
  pad_idx = jnp.full((e_pad - e,), n, jnp.int32)
  src3 = jnp.concatenate([edge_index[0], pad_idx]).reshape(nw, n_chunks, _CH)
  dst3 = jnp.concatenate([edge_index[1], pad_idx]).reshape(nw, n_chunks, _CH)

  (sums1,) = _sc_aggregate(n_pad, d, n_chunks)(x_pad, src3, dst3)
  (counts,) = _sc_counts(n_pad, d, n_chunks)(dst3)

  bn = n_pad // 4
  dense1 = _dense_call(_dense_layer_body, n_pad, bn, d, d, False)
  h1 = dense1(sums1[0], sums1[1], counts[0], counts[1], x_pad,
              Wl1, Wr1, b1[None, :])

  (sums2,) = _sc_aggregate(n_pad, d, n_chunks)(h1, src3, dst3)

  head = _dense_call(_dense_head_body, n_pad, bn, d, out_dim, True)
  out = head(sums2[0], sums2[1], counts[0], counts[1], h1,
             Wl2, Wr2, b2[None, :], Wout)
  return out[:n]
